# Initial kernel scaffold; baseline (speedup 1.0000x reference)
import functools
import math

import jax
import jax.numpy as jnp
from jax import lax
from jax.experimental import pallas as pl
from jax.experimental.pallas import tpu as pltpu
from jax.experimental.pallas import tpu_sc as plsc

N_NODES = 50000
N_EDGES = 800000
N_FEATURES = 128
N_BASIS = 16
R_CUT = 5.0

NW = 32

ROWS_PER_W = 1568
ROW_CHUNK = 224
N_ROW_CHUNKS = ROWS_PER_W // ROW_CHUNK

EDGES_PER_TILE = 25000
EDGES_PER_W = 25008
EDGE_CHUNK = 8336
N_EDGE_CHUNKS = EDGES_PER_W // EDGE_CHUNK
VECS_PER_CHUNK = EDGE_CHUNK // 16


def _sc_body(z_hbm, pos_t_hbm, edge_hbm, emb_hbm, atom_hbm, dxyz_hbm,
             idx_v, rows_v, pcomp_v, src_v, dst_v, out_v, sem):
    cid = lax.axis_index("c")
    sid = lax.axis_index("s")
    wid = sid * 2 + cid

    row0 = jnp.minimum(wid * ROWS_PER_W, N_NODES - ROWS_PER_W)
    for k in range(N_ROW_CHUNKS):
        base = row0 + k * ROW_CHUNK
        pltpu.sync_copy(z_hbm.at[pl.ds(base, ROW_CHUNK)], idx_v)
        pltpu.async_copy(emb_hbm.at[idx_v], rows_v, sem).wait()
        pltpu.sync_copy(rows_v, atom_hbm.at[pl.ds(base, ROW_CHUNK)])

    eb = jnp.minimum(wid * EDGES_PER_TILE, N_EDGES - EDGES_PER_W)
    for c in range(3):
        pltpu.sync_copy(pos_t_hbm.at[c], pcomp_v)
        for k in range(N_EDGE_CHUNKS):
            e0 = eb + k * EDGE_CHUNK
            pltpu.sync_copy(edge_hbm.at[0, pl.ds(e0, EDGE_CHUNK)], src_v)
            pltpu.sync_copy(edge_hbm.at[1, pl.ds(e0, EDGE_CHUNK)], dst_v)

            def body(i, carry):
                s = src_v[pl.ds(i * 16, 16)]
                t = dst_v[pl.ds(i * 16, 16)]
                a = plsc.load_gather(pcomp_v, [s])
                b = plsc.load_gather(pcomp_v, [t])
                out_v[pl.ds(i * 16, 16)] = a - b
                return carry

            lax.fori_loop(0, VECS_PER_CHUNK, body, 0)
            pltpu.sync_copy(out_v, dxyz_hbm.at[c, pl.ds(e0, EDGE_CHUNK)])


_sc_kernel = functools.partial(
    pl.kernel,
    out_type=(
        jax.ShapeDtypeStruct((N_NODES, N_FEATURES), jnp.float32),
        jax.ShapeDtypeStruct((3, N_EDGES), jnp.float32),
    ),
    mesh=plsc.VectorSubcoreMesh(core_axis_name="c", subcore_axis_name="s"),
    scratch_types=[
        pltpu.VMEM((ROW_CHUNK,), jnp.int32),
        pltpu.VMEM((ROW_CHUNK, N_FEATURES), jnp.float32),
        pltpu.VMEM((N_NODES,), jnp.float32),
        pltpu.VMEM((EDGE_CHUNK,), jnp.int32),
        pltpu.VMEM((EDGE_CHUNK,), jnp.int32),
        pltpu.VMEM((EDGE_CHUNK,), jnp.float32),
        pltpu.SemaphoreType.DMA,
    ],
)(_sc_body)


_EDGE_BLK = 4000
_N_EDGE_BLKS = N_EDGES // _EDGE_BLK
_CENTER_STEP = R_CUT / (N_BASIS - 1)
_GAMMA = 1.0 / (_CENTER_STEP * _CENTER_STEP)


def _tc_body(disp_ref, out_ref):
    d = disp_ref[...]
    d2 = d[:, 0:1] * d[:, 0:1] + d[:, 1:2] * d[:, 1:2] + d[:, 2:3] * d[:, 2:3]
    dist = jnp.sqrt(d2)
    cut = 0.5 * (jnp.cos(dist * (math.pi / R_CUT)) + 1.0)
    cut = cut * (dist < R_CUT).astype(jnp.float32)
    centers = jnp.arange(N_BASIS, dtype=jnp.float32)[None, :] * _CENTER_STEP
    delta = dist - centers
    out_ref[...] = cut * jnp.exp(-_GAMMA * delta * delta)


def _dist_edge(disp_edge):
    return pl.pallas_call(
        _tc_body,
        grid=(_N_EDGE_BLKS,),
        in_specs=[pl.BlockSpec((_EDGE_BLK, 3), lambda i: (i, 0))],
        out_specs=pl.BlockSpec((_EDGE_BLK, N_BASIS), lambda i: (i, 0)),
        out_shape=jax.ShapeDtypeStruct((N_EDGES, N_BASIS), jnp.float32),
    )(disp_edge)


def kernel(z, pos, edge_index, emb_table):
    pos_t = pos.T
    atom_node, dxyz = _sc_kernel(z, pos_t, edge_index, emb_table)
    disp_edge = dxyz.T
    dist_edge = _dist_edge(disp_edge)
    zeros = jnp.zeros((N_NODES, 3, N_FEATURES), dtype=jnp.float32)
    return (atom_node, zeros, zeros, disp_edge, dist_edge)


# speedup vs baseline: 3.6156x; 3.6156x over previous
import functools
import math

import jax
import jax.numpy as jnp
from jax import lax
from jax.experimental import pallas as pl
from jax.experimental.pallas import tpu as pltpu
from jax.experimental.pallas import tpu_sc as plsc

N_NODES = 50000
N_EDGES = 800000
N_FEATURES = 128
N_BASIS = 16
R_CUT = 5.0

NW = 32

ROWS_PER_W = 1568
ROW_CHUNK = 224
N_ROW_CHUNKS = ROWS_PER_W // ROW_CHUNK

EDGES_PER_TILE = 25000
EDGES_PER_W = 25008
EDGE_CHUNK = 8336
N_EDGE_CHUNKS = EDGES_PER_W // EDGE_CHUNK
VECS_PER_CHUNK = EDGE_CHUNK // 16


def _sc_body(z_hbm, px_hbm, py_hbm, pz_hbm, src_hbm, dst_hbm, emb_hbm,
             atom_hbm, dx_hbm, dy_hbm, dz_hbm,
             idx_v, rows_v, pcomp_v, src_v, dst_v, out_v, sem):
    cid = lax.axis_index("c")
    sid = lax.axis_index("s")
    wid = sid * 2 + cid

    row0 = jnp.minimum(wid * ROWS_PER_W, N_NODES - ROWS_PER_W)
    for k in range(N_ROW_CHUNKS):
        base = row0 + k * ROW_CHUNK
        pltpu.sync_copy(z_hbm.at[pl.ds(base, ROW_CHUNK)], idx_v)
        pltpu.async_copy(emb_hbm.at[idx_v], rows_v, sem).wait()
        pltpu.sync_copy(rows_v, atom_hbm.at[pl.ds(base, ROW_CHUNK)])

    eb = jnp.minimum(wid * EDGES_PER_TILE, N_EDGES - EDGES_PER_W)
    for p_hbm, o_hbm in ((px_hbm, dx_hbm), (py_hbm, dy_hbm), (pz_hbm, dz_hbm)):
        pltpu.sync_copy(p_hbm, pcomp_v)
        for k in range(N_EDGE_CHUNKS):
            e0 = eb + k * EDGE_CHUNK
            pltpu.sync_copy(src_hbm.at[pl.ds(e0, EDGE_CHUNK)], src_v)
            pltpu.sync_copy(dst_hbm.at[pl.ds(e0, EDGE_CHUNK)], dst_v)

            def body(i, carry):
                s = src_v[pl.ds(i * 16, 16)]
                t = dst_v[pl.ds(i * 16, 16)]
                a = plsc.load_gather(pcomp_v, [s])
                b = plsc.load_gather(pcomp_v, [t])
                out_v[pl.ds(i * 16, 16)] = a - b
                return carry

            lax.fori_loop(0, VECS_PER_CHUNK, body, 0)
            pltpu.sync_copy(out_v, o_hbm.at[pl.ds(e0, EDGE_CHUNK)])


_sc_kernel = functools.partial(
    pl.kernel,
    out_type=(
        jax.ShapeDtypeStruct((N_NODES, N_FEATURES), jnp.float32),
        jax.ShapeDtypeStruct((N_EDGES,), jnp.float32),
        jax.ShapeDtypeStruct((N_EDGES,), jnp.float32),
        jax.ShapeDtypeStruct((N_EDGES,), jnp.float32),
    ),
    mesh=plsc.VectorSubcoreMesh(core_axis_name="c", subcore_axis_name="s"),
    compiler_params=pltpu.CompilerParams(needs_layout_passes=False),
    scratch_types=[
        pltpu.VMEM((ROW_CHUNK,), jnp.int32),
        pltpu.VMEM((ROW_CHUNK, N_FEATURES), jnp.float32),
        pltpu.VMEM((N_NODES,), jnp.float32),
        pltpu.VMEM((EDGE_CHUNK,), jnp.int32),
        pltpu.VMEM((EDGE_CHUNK,), jnp.int32),
        pltpu.VMEM((EDGE_CHUNK,), jnp.float32),
        pltpu.SemaphoreType.DMA,
    ],
)(_sc_body)


_EDGE_BLK = 6400
_ROWS = _EDGE_BLK // 8
_N_EDGE_BLKS = N_EDGES // _EDGE_BLK
_CENTER_STEP = R_CUT / (N_BASIS - 1)
_GAMMA = 1.0 / (_CENTER_STEP * _CENTER_STEP)


def _tc_body(dx_ref, dy_ref, dz_ref, out_ref):
    lane = lax.broadcasted_iota(jnp.int32, (8, 128), 1)
    rep = (lane // N_BASIS == lax.broadcasted_iota(jnp.int32, (8, 128), 0))
    rep = rep.astype(jnp.float32)
    x = jnp.dot(dx_ref[...], rep, preferred_element_type=jnp.float32,
                precision=lax.Precision.HIGHEST)
    y = jnp.dot(dy_ref[...], rep, preferred_element_type=jnp.float32,
                precision=lax.Precision.HIGHEST)
    z = jnp.dot(dz_ref[...], rep, preferred_element_type=jnp.float32,
                precision=lax.Precision.HIGHEST)
    d2 = x * x + y * y + z * z
    dist = jnp.sqrt(d2)
    cut = 0.5 * (jnp.cos(dist * (math.pi / R_CUT)) + 1.0)
    cut = cut * (dist < R_CUT).astype(jnp.float32)
    centers = (lax.broadcasted_iota(jnp.int32, (_ROWS, 128), 1) % N_BASIS
               ).astype(jnp.float32) * _CENTER_STEP
    delta = dist - centers
    out_ref[...] = cut * jnp.exp(-_GAMMA * delta * delta)


def _dist_edge(dx, dy, dz):
    spec = pl.BlockSpec((_ROWS, 8), lambda i: (i, 0))
    flat = pl.pallas_call(
        _tc_body,
        grid=(_N_EDGE_BLKS,),
        in_specs=[spec, spec, spec],
        out_specs=pl.BlockSpec((_ROWS, 128), lambda i: (i, 0)),
        out_shape=jax.ShapeDtypeStruct((N_EDGES // 8, 128), jnp.float32),
    )(dx.reshape(N_EDGES // 8, 8), dy.reshape(N_EDGES // 8, 8),
      dz.reshape(N_EDGES // 8, 8))
    return flat.reshape(N_EDGES, N_BASIS)


def kernel(z, pos, edge_index, emb_table):
    px, py, pz = pos[:, 0], pos[:, 1], pos[:, 2]
    src, dst = edge_index[0], edge_index[1]
    atom_node, dx, dy, dz = _sc_kernel(z, px, py, pz, src, dst, emb_table)
    disp_edge = jnp.stack([dx, dy, dz], axis=1)
    dist_edge = _dist_edge(dx, dy, dz)
    zeros = jnp.zeros((N_NODES, 3, N_FEATURES), dtype=jnp.float32)
    return (atom_node, zeros, zeros, disp_edge, dist_edge)


# speedup vs baseline: 4.7861x; 1.3237x over previous
import functools
import math

import jax
import jax.numpy as jnp
from jax import lax
from jax.experimental import pallas as pl
from jax.experimental.pallas import tpu as pltpu
from jax.experimental.pallas import tpu_sc as plsc

N_NODES = 50000
N_EDGES = 800000
N_FEATURES = 128
N_BASIS = 16
R_CUT = 5.0

NW = 32

ROWS_PER_W = 1568
ROW_CHUNK = 224
N_ROW_CHUNKS = ROWS_PER_W // ROW_CHUNK

EDGES_PER_TILE = 25000
EDGES_PER_W = 25008
EDGE_CHUNK = 8336
N_EDGE_CHUNKS = EDGES_PER_W // EDGE_CHUNK
VECS_PER_CHUNK = EDGE_CHUNK // 16


def _sc_body(z_hbm, px_hbm, py_hbm, pz_hbm, src_hbm, dst_hbm, emb_hbm,
             atom_hbm, dx_hbm, dy_hbm, dz_hbm,
             idx_v, rows_v, pcomp_v, src_v, dst_v, out_v, sem):
    cid = lax.axis_index("c")
    sid = lax.axis_index("s")
    wid = sid * 2 + cid

    row0 = jnp.minimum(wid * ROWS_PER_W, N_NODES - ROWS_PER_W)
    for k in range(N_ROW_CHUNKS):
        base = row0 + k * ROW_CHUNK
        pltpu.sync_copy(z_hbm.at[pl.ds(base, ROW_CHUNK)], idx_v)
        pltpu.async_copy(emb_hbm.at[idx_v], rows_v, sem).wait()
        pltpu.sync_copy(rows_v, atom_hbm.at[pl.ds(base, ROW_CHUNK)])

    eb = jnp.minimum(wid * EDGES_PER_TILE, N_EDGES - EDGES_PER_W)
    for p_hbm, o_hbm in ((px_hbm, dx_hbm), (py_hbm, dy_hbm), (pz_hbm, dz_hbm)):
        pltpu.sync_copy(p_hbm, pcomp_v)
        for k in range(N_EDGE_CHUNKS):
            e0 = eb + k * EDGE_CHUNK
            pltpu.sync_copy(src_hbm.at[pl.ds(e0, EDGE_CHUNK)], src_v)
            pltpu.sync_copy(dst_hbm.at[pl.ds(e0, EDGE_CHUNK)], dst_v)

            def body(i, carry):
                s = src_v[pl.ds(i * 16, 16)]
                t = dst_v[pl.ds(i * 16, 16)]
                a = plsc.load_gather(pcomp_v, [s])
                b = plsc.load_gather(pcomp_v, [t])
                out_v[pl.ds(i * 16, 16)] = a - b
                return carry

            lax.fori_loop(0, VECS_PER_CHUNK, body, 0)
            pltpu.sync_copy(out_v, o_hbm.at[pl.ds(e0, EDGE_CHUNK)])


_sc_kernel = functools.partial(
    pl.kernel,
    out_type=(
        jax.ShapeDtypeStruct((N_NODES, N_FEATURES), jnp.float32),
        jax.ShapeDtypeStruct((N_EDGES,), jnp.float32),
        jax.ShapeDtypeStruct((N_EDGES,), jnp.float32),
        jax.ShapeDtypeStruct((N_EDGES,), jnp.float32),
    ),
    mesh=plsc.VectorSubcoreMesh(core_axis_name="c", subcore_axis_name="s"),
    compiler_params=pltpu.CompilerParams(needs_layout_passes=False),
    scratch_types=[
        pltpu.VMEM((ROW_CHUNK,), jnp.int32),
        pltpu.VMEM((ROW_CHUNK, N_FEATURES), jnp.float32),
        pltpu.VMEM((N_NODES,), jnp.float32),
        pltpu.VMEM((EDGE_CHUNK,), jnp.int32),
        pltpu.VMEM((EDGE_CHUNK,), jnp.int32),
        pltpu.VMEM((EDGE_CHUNK,), jnp.float32),
        pltpu.SemaphoreType.DMA,
    ],
)(_sc_body)


_EDGE_BLK = 6400
_ROWS = _EDGE_BLK // 8
_N_EDGE_BLKS = N_EDGES // _EDGE_BLK
_CENTER_STEP = R_CUT / (N_BASIS - 1)
_GAMMA = 1.0 / (_CENTER_STEP * _CENTER_STEP)


_COS_POLY = (0.9999999890590233, -4.934801124863485, 4.058694841243486,
             -1.3351584301699686, 0.23502980840174797,
             -0.025358983640522026, 0.001593910683660976)


def _tc_dense_body(dx_ref, dy_ref, dz_ref, d_ref, cut_ref):
    x = dx_ref[...]
    y = dy_ref[...]
    z = dz_ref[...]
    d2 = x * x + y * y + z * z
    dist = jnp.sqrt(d2)
    v = d2 * (1.0 / (R_CUT * R_CUT))
    p = jnp.float32(_COS_POLY[6])
    for coef in _COS_POLY[5::-1]:
        p = p * v + jnp.float32(coef)
    cut = 0.5 * (p + 1.0)
    d_ref[...] = dist
    cut_ref[...] = cut * (v < 1.0).astype(jnp.float32)


def _tc_rep_body(d8_ref, cut8_ref, out_ref):
    idx = lax.broadcasted_iota(jnp.int32, (_ROWS, 128), 1) // N_BASIS
    dist = jnp.take_along_axis(d8_ref[...], idx, axis=1)
    cut = jnp.take_along_axis(cut8_ref[...], idx, axis=1)
    centers = (lax.broadcasted_iota(jnp.int32, (_ROWS, 128), 1) % N_BASIS
               ).astype(jnp.float32) * _CENTER_STEP
    delta = dist - centers
    out_ref[...] = cut * jnp.exp(-_GAMMA * delta * delta)


_DENSE_ROWS = N_EDGES // 128
_DENSE_BLK = 800
_N_DENSE_BLKS = -(-_DENSE_ROWS // _DENSE_BLK)


def _dist_edge(dx, dy, dz):
    dspec = pl.BlockSpec((_DENSE_BLK, 128), lambda i: (i, 0))
    dist, cut = pl.pallas_call(
        _tc_dense_body,
        grid=(_N_DENSE_BLKS,),
        in_specs=[dspec, dspec, dspec],
        out_specs=[dspec, dspec],
        out_shape=[jax.ShapeDtypeStruct((_DENSE_ROWS, 128), jnp.float32),
                   jax.ShapeDtypeStruct((_DENSE_ROWS, 128), jnp.float32)],
    )(dx.reshape(_DENSE_ROWS, 128), dy.reshape(_DENSE_ROWS, 128),
      dz.reshape(_DENSE_ROWS, 128))
    spec = pl.BlockSpec((_ROWS, 8), lambda i: (i, 0))
    flat = pl.pallas_call(
        _tc_rep_body,
        grid=(_N_EDGE_BLKS,),
        in_specs=[spec, spec],
        out_specs=pl.BlockSpec((_ROWS, 128), lambda i: (i, 0)),
        out_shape=jax.ShapeDtypeStruct((N_EDGES // 8, 128), jnp.float32),
    )(dist.reshape(N_EDGES // 8, 8), cut.reshape(N_EDGES // 8, 8))
    return flat.reshape(N_EDGES, N_BASIS)


def kernel(z, pos, edge_index, emb_table):
    px, py, pz = pos[:, 0], pos[:, 1], pos[:, 2]
    src, dst = edge_index[0], edge_index[1]
    atom_node, dx, dy, dz = _sc_kernel(z, px, py, pz, src, dst, emb_table)
    disp_edge = jnp.stack([dx, dy, dz], axis=1)
    dist_edge = _dist_edge(dx, dy, dz)
    zeros = jnp.zeros((N_NODES, 3, N_FEATURES), dtype=jnp.float32)
    return (atom_node, zeros, zeros, disp_edge, dist_edge)
